# baseline (speedup 1.0000x reference)
import jax
import jax.numpy as jnp
from jax.experimental import pallas as pl

SEQ_BLOCK = 512


def _add_body(x_ref, pe_ref, o_ref):
    o_ref[...] = x_ref[...] + pe_ref[...]


def kernel(x, pe):
    b, s, d = x.shape
    pe_s = pe[:, :s, :]
    n_seq = s // SEQ_BLOCK
    return pl.pallas_call(
        _add_body,
        grid=(n_seq,),
        in_specs=[
            pl.BlockSpec((b, SEQ_BLOCK, d), lambda i: (0, i, 0)),
            pl.BlockSpec((1, SEQ_BLOCK, d), lambda i: (0, i, 0)),
        ],
        out_specs=pl.BlockSpec((b, SEQ_BLOCK, d), lambda i: (0, i, 0)),
        out_shape=jax.ShapeDtypeStruct((b, s, d), x.dtype),
    )(x, pe_s)
